# Initial kernel scaffold; baseline (speedup 1.0000x reference)
import jax
import jax.numpy as jnp
from jax.experimental import pallas as pl


def kernel(x, edge_index, lin0_w, lin0_b, lin1_w, lin1_b, mlp_w1, mlp_b1, mlp_w2, mlp_b2, mlp_w3, mlp_b3, parsing0, parsing1, conv0_w1, conv0_w2, conv1_w1, conv1_w2):
    raise NotImplementedError("write your pallas kernel here")


# speedup vs baseline: 3.7764x; 3.7764x over previous
import functools
import math

import jax
import jax.numpy as jnp
from jax import lax
from jax.experimental import pallas as pl
from jax.experimental.pallas import tpu as pltpu
from jax.experimental.pallas import tpu_sc as plsc

_N = 10000
_E = 320000
_D = 128
_ALPHA = 0.0
_THETA = 1.0
_SCALE = 2.0
_EPSV = 0.0001

_NC = 2
_NS = 16
_NW = _NC * _NS
_NPAD = 10240
_ROWS_PER_TILE = _NPAD // _NS
_EDGES_PER_W = _E // _NW
_K = 400
_NCHUNK = _EDGES_PER_W // _K

@functools.cache
def _sc_mesh():
    return plsc.VectorSubcoreMesh(
        core_axis_name="c", subcore_axis_name="s", num_cores=_NC, num_subcores=_NS
    )


_DH = _D // 2


def _conv_agg_body(g0_h, g1_h, row_h, col_h, ew_h, out_h, ridx, cidx, ewv, rows,
                   acc, sem):
    cid = lax.axis_index("c")
    sid = lax.axis_index("s")
    wid = sid * _NC + cid
    base_rows = sid * _ROWS_PER_TILE
    ebase = wid * _EDGES_PER_W

    for f, g_h in enumerate((g0_h, g1_h)):
        zvec = jnp.zeros((16,), jnp.float32)

        def _zero_rows(i, _):
            for j in range(_DH // 16):
                rows[i, pl.ds(j * 16, 16)] = zvec
            return 0

        lax.fori_loop(0, _K, _zero_rows, 0)
        pltpu.sync_copy(rows.at[pl.ds(0, _K)], acc.at[pl.ds(base_rows, _K)])
        pltpu.sync_copy(
            rows.at[pl.ds(0, _ROWS_PER_TILE - _K)],
            acc.at[pl.ds(base_rows + _K, _ROWS_PER_TILE - _K)],
        )
        plsc.subcore_barrier()

        def _chunk(c, _):
            off = ebase + c * _K
            pltpu.sync_copy(row_h.at[pl.ds(off, _K)], ridx)
            pltpu.sync_copy(col_h.at[pl.ds(off, _K)], cidx)
            pltpu.sync_copy(ew_h.at[pl.ds(off, _K)], ewv)
            pltpu.async_copy(g_h.at[ridx], rows, sem).wait()

            def _scale(k16, _):
                kbase = k16 * 16
                wv = ewv[pl.ds(kbase, 16)]
                for l in range(16):
                    w = wv[l]
                    for j in range(_DH // 16):
                        sl = pl.ds(j * 16, 16)
                        rows[kbase + l, sl] = rows[kbase + l, sl] * w
                return 0

            lax.fori_loop(0, _K // 16, _scale, 0)
            pltpu.sync_copy(rows, acc.at[cidx], add=True)
            return 0

        lax.fori_loop(0, _NCHUNK, _chunk, 0)
        plsc.subcore_barrier()

        pltpu.sync_copy(
            acc.at[pl.ds(base_rows, _ROWS_PER_TILE)],
            out_h.at[cid, f, pl.ds(base_rows, _ROWS_PER_TILE)],
        )
        if f == 0:
            plsc.subcore_barrier()


@jax.jit
def _conv_agg(g0, g1, row, col, ew):
    k = pl.kernel(
        _conv_agg_body,
        out_type=jax.ShapeDtypeStruct((_NC, 2, _NPAD, _DH), jnp.float32),
        mesh=_sc_mesh(),
        compiler_params=pltpu.CompilerParams(use_tc_tiling_on_sc=False),
        scratch_types=[
            pltpu.VMEM((_K,), jnp.int32),
            pltpu.VMEM((_K,), jnp.int32),
            pltpu.VMEM((_K,), jnp.float32),
            pltpu.VMEM((_K, _DH), jnp.float32),
            pltpu.VMEM_SHARED((_NPAD, _DH), jnp.float32),
            pltpu.SemaphoreType.DMA,
        ],
    )
    return k(g0, g1, row, col, ew)


def kernel(x, edge_index, lin0_w, lin0_b, lin1_w, lin1_b, mlp_w1, mlp_b1,
           mlp_w2, mlp_b2, mlp_w3, mlp_b3, parsing0, parsing1, conv0_w1,
           conv0_w2, conv1_w1, conv1_w2):
    row = edge_index[0]
    col = edge_index[1]

    h = jax.nn.relu(x @ mlp_w1 + mlp_b1)
    h = jax.nn.relu(h @ mlp_w2 + mlp_b2)
    pl_feat = h @ mlp_w3 + mlp_b3
    P = jax.nn.relu(_SCALE * parsing0)
    v_feat = pl_feat @ P
    raw = jnp.sum(pl_feat[row] * v_feat[col], axis=1)
    mean = raw.mean()
    var = jnp.var(raw, ddof=1)
    ew = (raw - mean) * jnp.sqrt(_EPSV / var) + 1.0

    deg = jnp.ones((_N,), jnp.float32).at[col].add(ew)
    dinv = jnp.where(deg > 0, lax.rsqrt(jnp.where(deg > 0, deg, 1.0)), 0.0)

    h = jax.nn.relu(x @ lin0_w + lin0_b)
    conv_ws = [(conv0_w1, conv0_w2), (conv1_w1, conv1_w2)]
    for ln in range(2):
        beta = math.log(_THETA / (ln + 1) + 1.0)
        g = h * dinv[:, None]
        g_pad = jnp.zeros((_NPAD, _D), jnp.float32).at[: _N].set(g)
        parts = _conv_agg(g_pad[:, : _DH], g_pad[:, _DH:], row, col, ew)
        parts = jnp.concatenate([parts[:, 0], parts[:, 1]], axis=-1)
        agg = (parts[0, : _N] + parts[1, : _N]) * dinv[:, None]
        agg = agg + (dinv * dinv)[:, None] * h
        out = (1.0 - beta) * agg + beta * (agg @ conv_ws[ln][0])
        h = jax.nn.relu(out)

    return h @ lin1_w + lin1_b


# speedup vs baseline: 9.4178x; 2.4939x over previous
import functools
import math

import jax
import jax.numpy as jnp
from jax import lax
from jax.experimental import pallas as pl
from jax.experimental.pallas import tpu as pltpu
from jax.experimental.pallas import tpu_sc as plsc

_N = 10000
_E = 320000
_D = 128
_ALPHA = 0.0
_THETA = 1.0
_SCALE = 2.0
_EPSV = 0.0001

_NC = 2
_NS = 16
_NW = _NC * _NS
_NPAD = 10240
_ROWS_PER_TILE = _NPAD // _NS
_EDGES_PER_W = _E // _NW
_K = 400
_NCHUNK = _EDGES_PER_W // _K

@functools.cache
def _sc_mesh():
    return plsc.VectorSubcoreMesh(
        core_axis_name="c", subcore_axis_name="s", num_cores=_NC, num_subcores=_NS
    )


_DH = _D // 2
_NPAIR = (_NCHUNK - 1) // 2


def _conv_agg_body(g0_h, g1_h, edata_h, ab_h, out_h, ebuf, rows0, rows1, cidx0,
                   cidx1, abuf, acc, sg0, sg1, ss0, ss1):
    cid = lax.axis_index("c")
    sid = lax.axis_index("s")
    wid = sid * _NC + cid
    base_rows = sid * _ROWS_PER_TILE
    rows = (rows0, rows1)
    cidx = (cidx0, cidx1)
    gsem = (sg0, sg1)
    ssem = (ss0, ss1)
    zvec = jnp.zeros((16,), jnp.float32)

    pltpu.sync_copy(edata_h.at[pl.ds(wid * _NCHUNK, _NCHUNK)], ebuf)
    pltpu.sync_copy(ab_h, abuf)
    abv = abuf[pl.ds(0, 16)]
    a_s = abv[0]
    b_s = abv[1]

    for f in range(2):
        g_h = (g0_h, g1_h)[f]

        def _zero(i, _):
            for j in range(_DH // 16):
                rows0[i, pl.ds(j * 16, 16)] = zvec
            return 0

        lax.fori_loop(0, _K, _zero, 0)
        pltpu.sync_copy(rows0.at[pl.ds(0, _K)], acc.at[pl.ds(base_rows, _K)])
        pltpu.sync_copy(
            rows0.at[pl.ds(0, _ROWS_PER_TILE - _K)],
            acc.at[pl.ds(base_rows + _K, _ROWS_PER_TILE - _K)],
        )
        plsc.subcore_barrier()

        def _scale(c, b):
            def body(k16, _):
                kb = k16 * 16
                ew = plsc.bitcast(ebuf[c, 2, pl.ds(kb, 16)], jnp.float32)
                ew = ew * a_s + b_s
                cidx[b][pl.ds(kb, 16)] = ebuf[c, 1, pl.ds(kb, 16)]
                for l in range(16):
                    w = ew[l]
                    for j in range(_DH // 16):
                        sl = pl.ds(j * 16, 16)
                        rows[b][kb + l, sl] = rows[b][kb + l, sl] * w
                return 0

            lax.fori_loop(0, _K // 16, body, 0)

        pltpu.async_copy(g_h.at[ebuf.at[0, 0]], rows0, sg0)
        pltpu.make_async_copy(g_h.at[ebuf.at[0, 0]], rows0, sg0).wait()
        _scale(0, 0)
        pltpu.async_copy(g_h.at[ebuf.at[1, 0]], rows1, sg1)
        pltpu.async_copy(rows0, acc.at[cidx0], ss0, add=True)

        def _pair(c2, _):
            for b2 in range(2):
                c = 1 + 2 * c2 + b2
                b = 1 - b2
                nb = 1 - b
                pltpu.make_async_copy(g_h.at[ebuf.at[c, 0]], rows[b],
                                      gsem[b]).wait()
                _scale(c, b)
                pltpu.make_async_copy(rows[nb], acc.at[cidx[nb]],
                                      ssem[nb]).wait()
                if b2 == 0:
                    pltpu.async_copy(g_h.at[ebuf.at[c + 1, 0]], rows[nb],
                                     gsem[nb])
                else:
                    @pl.when(c2 < _NPAIR - 1)
                    def _start_next():
                        pltpu.async_copy(g_h.at[ebuf.at[c + 1, 0]], rows[nb],
                                         gsem[nb])
                pltpu.async_copy(rows[b], acc.at[cidx[b]], ssem[b], add=True)
            return 0

        lax.fori_loop(0, _NPAIR, _pair, 0)
        pltpu.make_async_copy(rows0, acc.at[cidx0], ss0).wait()
        plsc.subcore_barrier()

        pltpu.sync_copy(
            acc.at[pl.ds(base_rows, _ROWS_PER_TILE)],
            out_h.at[cid, f, pl.ds(base_rows, _ROWS_PER_TILE)],
        )
        if f == 0:
            plsc.subcore_barrier()


@jax.jit
def _conv_agg(g0, g1, edata, ab16):
    k = pl.kernel(
        _conv_agg_body,
        out_type=jax.ShapeDtypeStruct((_NC, 2, _NPAD, _DH), jnp.float32),
        mesh=_sc_mesh(),
        compiler_params=pltpu.CompilerParams(
            use_tc_tiling_on_sc=False, needs_layout_passes=False
        ),
        scratch_types=[
            pltpu.VMEM((_NCHUNK, 3, _K), jnp.int32),
            pltpu.VMEM((_K, _DH), jnp.float32),
            pltpu.VMEM((_K, _DH), jnp.float32),
            pltpu.VMEM((_K,), jnp.int32),
            pltpu.VMEM((_K,), jnp.int32),
            pltpu.VMEM((16,), jnp.float32),
            pltpu.VMEM_SHARED((_NPAD, _DH), jnp.float32),
            pltpu.SemaphoreType.DMA,
            pltpu.SemaphoreType.DMA,
            pltpu.SemaphoreType.DMA,
            pltpu.SemaphoreType.DMA,
        ],
    )
    return k(g0, g1, edata, ab16)


def _edge_w_body(u_h, v_h, eidx_h, raw_h, sc_h, part_h, ebuf, su, sv, cidxb,
                 rawb, onesb, zb, pbuf, S_acc, C_acc, semu, semv):
    cid = lax.axis_index("c")
    sid = lax.axis_index("s")
    wid = sid * _NC + cid
    base_rows = sid * _ROWS_PER_TILE
    zvec = jnp.zeros((16,), jnp.float32)
    ovec = jnp.ones((16,), jnp.float32)
    lane = lax.iota(jnp.int32, 16)

    pltpu.sync_copy(eidx_h.at[pl.ds(wid * _NCHUNK, _NCHUNK)], ebuf)

    def _fill(i, _):
        zb[pl.ds(i * 16, 16)] = zvec
        onesb[pl.ds(i * 16, 16)] = ovec
        return 0

    lax.fori_loop(0, _K // 16, _fill, 0)

    def _fillz(i, _):
        zb[pl.ds(i * 16, 16)] = zvec
        return 0

    lax.fori_loop(_K // 16, _ROWS_PER_TILE // 16, _fillz, 0)
    pltpu.sync_copy(zb, S_acc.at[pl.ds(base_rows, _ROWS_PER_TILE)])
    pltpu.sync_copy(zb, C_acc.at[pl.ds(base_rows, _ROWS_PER_TILE)])
    plsc.subcore_barrier()

    def _chunk(c, carry):
        vsum, vsq = carry
        pltpu.async_copy(u_h.at[ebuf.at[c, 0]], su, semu)
        pltpu.async_copy(v_h.at[ebuf.at[c, 1]], sv, semv)
        pltpu.make_async_copy(u_h.at[ebuf.at[c, 0]], su, semu).wait()
        pltpu.make_async_copy(v_h.at[ebuf.at[c, 1]], sv, semv).wait()

        def _dot16(k16, carry2):
            vs, vq = carry2
            kb = k16 * 16
            cidxb[pl.ds(kb, 16)] = ebuf[c, 1, pl.ds(kb, 16)]
            rawv = zvec
            for l in range(16):
                m = su[kb + l, pl.ds(0, 16)] * sv[kb + l, pl.ds(0, 16)]
                s = jnp.sum(m)
                rawv = jnp.where(lane == l, s, rawv)
            rawb[pl.ds(kb, 16)] = rawv
            return (vs + rawv, vq + rawv * rawv)

        carry = lax.fori_loop(0, _K // 16, _dot16, (vsum, vsq))
        off = wid * _EDGES_PER_W + c * _K
        pltpu.sync_copy(rawb, raw_h.at[pl.ds(off, _K)])
        pltpu.sync_copy(rawb, S_acc.at[cidxb], add=True)
        pltpu.sync_copy(onesb, C_acc.at[cidxb], add=True)
        return carry

    vsum, vsq = lax.fori_loop(0, _NCHUNK, _chunk, (zvec, zvec))
    plsc.subcore_barrier()

    pltpu.sync_copy(S_acc.at[pl.ds(base_rows, _ROWS_PER_TILE)],
                    sc_h.at[cid, 0, pl.ds(base_rows, _ROWS_PER_TILE)])
    pltpu.sync_copy(C_acc.at[pl.ds(base_rows, _ROWS_PER_TILE)],
                    sc_h.at[cid, 1, pl.ds(base_rows, _ROWS_PER_TILE)])
    pbuf[0, pl.ds(0, 16)] = vsum
    pbuf[1, pl.ds(0, 16)] = vsq
    pltpu.sync_copy(pbuf, part_h.at[wid])


@jax.jit
def _edge_w(u16, v16, eidx):
    k = pl.kernel(
        _edge_w_body,
        out_type=(
            jax.ShapeDtypeStruct((_E,), jnp.float32),
            jax.ShapeDtypeStruct((_NC, 2, _NPAD), jnp.float32),
            jax.ShapeDtypeStruct((_NW, 2, 16), jnp.float32),
        ),
        mesh=_sc_mesh(),
        compiler_params=pltpu.CompilerParams(
            use_tc_tiling_on_sc=False, needs_layout_passes=False
        ),
        scratch_types=[
            pltpu.VMEM((_NCHUNK, 2, _K), jnp.int32),
            pltpu.VMEM((_K, 16), jnp.float32),
            pltpu.VMEM((_K, 16), jnp.float32),
            pltpu.VMEM((_K,), jnp.int32),
            pltpu.VMEM((_K,), jnp.float32),
            pltpu.VMEM((_K,), jnp.float32),
            pltpu.VMEM((_ROWS_PER_TILE,), jnp.float32),
            pltpu.VMEM((2, 16), jnp.float32),
            pltpu.VMEM_SHARED((_NPAD,), jnp.float32),
            pltpu.VMEM_SHARED((_NPAD,), jnp.float32),
            pltpu.SemaphoreType.DMA,
            pltpu.SemaphoreType.DMA,
        ],
    )
    return k(u16, v16, eidx)


def _pack_eidx(row, col):
    ed = jnp.stack([row, col])
    ed = ed.reshape(2, _NW, _NCHUNK, _K).transpose(1, 2, 0, 3)
    return ed.reshape(_NW * _NCHUNK, 2, _K)


def _pack_edata(row, col, ew):
    ed = jnp.stack([row, col, lax.bitcast_convert_type(ew, jnp.int32)])
    ed = ed.reshape(3, _NW, _NCHUNK, _K).transpose(1, 2, 0, 3)
    return ed.reshape(_NW * _NCHUNK, 3, _K)


_BLK = 512
_GRID = _NPAD // _BLK


def _mm(a, b):
    return jnp.dot(a, b, precision=lax.Precision.HIGHEST)


def _full(shape):
    return pl.BlockSpec(shape, lambda i: tuple(0 for _ in shape))


def _rows(shape):
    return pl.BlockSpec(shape, lambda i: (i,) + tuple(0 for _ in shape[1:]))


@jax.jit
def _dense_pre(x_pad, mlp_w1, mlp_b1, mlp_w2, mlp_b2, mlp_w3, mlp_b3,
               parsing0, lin0_w, lin0_b):
    def body(x_ref, w1, b1, w2, b2, w3, b3, p0, l0w, l0b, u_ref, v_ref, h_ref):
        xb = x_ref[...]
        h1 = jnp.maximum(_mm(xb, w1[...]) + b1[...], 0.0)
        h2 = jnp.maximum(_mm(h1, w2[...]) + b2[...], 0.0)
        plv = _mm(h2, w3[...]) + b3[...]
        P = jnp.maximum(_SCALE * p0[...], 0.0)
        vv = _mm(plv, P)
        z = jnp.zeros((_BLK, 8), jnp.float32)
        u_ref[...] = jnp.concatenate([plv, z], axis=1)
        v_ref[...] = jnp.concatenate([vv, z], axis=1)
        h_ref[...] = jnp.maximum(_mm(xb, l0w[...]) + l0b[...], 0.0)

    return pl.pallas_call(
        body,
        grid=(_GRID,),
        in_specs=[
            _rows((_BLK, _D)),
            _full((_D, 512)), _full((1, 512)),
            _full((512, 64)), _full((1, 64)),
            _full((64, 8)), _full((1, 8)),
            _full((8, 8)),
            _full((_D, _D)), _full((1, _D)),
        ],
        out_specs=[_rows((_BLK, 16)), _rows((_BLK, 16)), _rows((_BLK, _D))],
        out_shape=[
            jax.ShapeDtypeStruct((_NPAD, 16), jnp.float32),
            jax.ShapeDtypeStruct((_NPAD, 16), jnp.float32),
            jax.ShapeDtypeStruct((_NPAD, _D), jnp.float32),
        ],
    )(x_pad, mlp_w1, mlp_b1.reshape(1, -1), mlp_w2, mlp_b2.reshape(1, -1),
      mlp_w3, mlp_b3.reshape(1, -1), parsing0, lin0_w, lin0_b.reshape(1, -1))


def _scale_split(h, dinv2):
    def body(h_ref, d_ref, g0_ref, g1_ref):
        g = h_ref[...] * d_ref[...][:, None]
        g0_ref[...] = g[:, : _DH]
        g1_ref[...] = g[:, _DH:]

    return pl.pallas_call(
        body,
        grid=(_GRID,),
        in_specs=[_rows((_BLK, _D)), _rows((_BLK,))],
        out_specs=[_rows((_BLK, _DH)), _rows((_BLK, _DH))],
        out_shape=[
            jax.ShapeDtypeStruct((_NPAD, _DH), jnp.float32),
            jax.ShapeDtypeStruct((_NPAD, _DH), jnp.float32),
        ],
    )(h, dinv2)


def _conv_mix(parts, dinv2, h, w1, beta):
    def body(p_ref, d_ref, h_ref, w_ref, o_ref):
        p = p_ref[...]
        agg = jnp.concatenate([p[0, 0] + p[1, 0], p[0, 1] + p[1, 1]], axis=1)
        dv = d_ref[...][:, None]
        agg = agg * dv + (dv * dv) * h_ref[...]
        out = (1.0 - beta) * agg + beta * _mm(agg, w_ref[...])
        o_ref[...] = jnp.maximum(out, 0.0)

    return pl.pallas_call(
        body,
        grid=(_GRID,),
        in_specs=[
            pl.BlockSpec((_NC, 2, _BLK, _DH), lambda i: (0, 0, i, 0)),
            _rows((_BLK,)),
            _rows((_BLK, _D)),
            _full((_D, _D)),
        ],
        out_specs=_rows((_BLK, _D)),
        out_shape=jax.ShapeDtypeStruct((_NPAD, _D), jnp.float32),
    )(parts, dinv2, h, w1)


def _final_proj(h, lin1_w, lin1_b):
    def body(h_ref, w_ref, b_ref, o_ref):
        o_ref[...] = _mm(h_ref[...], w_ref[...]) + b_ref[...]

    return pl.pallas_call(
        body,
        grid=(_GRID,),
        in_specs=[_rows((_BLK, _D)), _full((_D, 8)), _full((1, 8))],
        out_specs=_rows((_BLK, 8)),
        out_shape=jax.ShapeDtypeStruct((_NPAD, 8), jnp.float32),
    )(h, lin1_w, lin1_b.reshape(1, -1))


def kernel(x, edge_index, lin0_w, lin0_b, lin1_w, lin1_b, mlp_w1, mlp_b1,
           mlp_w2, mlp_b2, mlp_w3, mlp_b3, parsing0, parsing1, conv0_w1,
           conv0_w2, conv1_w1, conv1_w2):
    row = edge_index[0]
    col = edge_index[1]
    x_pad = jnp.zeros((_NPAD, _D), jnp.float32).at[: _N].set(x)

    u16, v16, h0 = _dense_pre(x_pad, mlp_w1, mlp_b1, mlp_w2, mlp_b2,
                              mlp_w3, mlp_b3, parsing0, lin0_w, lin0_b)

    eidx = _pack_eidx(row, col)
    raw, sc_parts, mv_parts = _edge_w(u16, v16, eidx)
    edata = _pack_edata(row, col, raw)
    S = sc_parts[0, 0] + sc_parts[1, 0]
    C = sc_parts[0, 1] + sc_parts[1, 1]
    sum_r = mv_parts[:, 0, :].sum()
    sum_q = mv_parts[:, 1, :].sum()
    mean = sum_r / _E
    var = (sum_q - _E * mean * mean) / (_E - 1)
    a = jnp.sqrt(_EPSV / var)
    b = 1.0 - a * mean
    ab16 = jnp.concatenate([a[None], b[None], jnp.zeros((14,), jnp.float32)])

    deg = 1.0 + a * S + b * C
    dinv = jnp.where(deg > 0, lax.rsqrt(jnp.where(deg > 0, deg, 1.0)), 0.0)
    dinv2 = dinv

    h = h0
    conv_ws = [conv0_w1, conv1_w1]
    for ln in range(2):
        beta = math.log(_THETA / (ln + 1) + 1.0)
        g0, g1 = _scale_split(h, dinv2)
        parts = _conv_agg(g0, g1, edata, ab16)
        h = _conv_mix(parts, dinv2, h, conv_ws[ln], beta)

    return _final_proj(h, lin1_w, lin1_b)[: _N]


# speedup vs baseline: 9.7606x; 1.0364x over previous
import functools
import math

import jax
import jax.numpy as jnp
from jax import lax
from jax.experimental import pallas as pl
from jax.experimental.pallas import tpu as pltpu
from jax.experimental.pallas import tpu_sc as plsc

_N = 10000
_E = 320000
_D = 128
_ALPHA = 0.0
_THETA = 1.0
_SCALE = 2.0
_EPSV = 0.0001

_NC = 2
_NS = 16
_NW = _NC * _NS
_NPAD = 10240
_ROWS_PER_TILE = _NPAD // _NS
_EDGES_PER_W = _E // _NW
_K = 400
_NCHUNK = _EDGES_PER_W // _K

@functools.cache
def _sc_mesh():
    return plsc.VectorSubcoreMesh(
        core_axis_name="c", subcore_axis_name="s", num_cores=_NC, num_subcores=_NS
    )


_DH = _D // 2
_NPAIR = (_NCHUNK - 1) // 2


def _conv_agg_body(g0_h, g1_h, edata_h, ab_h, out_h, ebuf, rows0, rows1, cidx0,
                   cidx1, abuf, acc, sg0, sg1, ss0, ss1):
    cid = lax.axis_index("c")
    sid = lax.axis_index("s")
    wid = sid * _NC + cid
    base_rows = sid * _ROWS_PER_TILE
    rows = (rows0, rows1)
    cidx = (cidx0, cidx1)
    gsem = (sg0, sg1)
    ssem = (ss0, ss1)
    zvec = jnp.zeros((16,), jnp.float32)

    pltpu.sync_copy(edata_h.at[pl.ds(wid * _NCHUNK, _NCHUNK)], ebuf)
    pltpu.sync_copy(ab_h, abuf)
    abv = abuf[pl.ds(0, 16)]
    a_s = abv[0]
    b_s = abv[1]

    for f in range(2):
        g_h = (g0_h, g1_h)[f]

        def _zero(i, _):
            for j in range(_DH // 16):
                rows0[i, pl.ds(j * 16, 16)] = zvec
            return 0

        lax.fori_loop(0, _K, _zero, 0)
        pltpu.sync_copy(rows0.at[pl.ds(0, _K)], acc.at[pl.ds(base_rows, _K)])
        pltpu.sync_copy(
            rows0.at[pl.ds(0, _ROWS_PER_TILE - _K)],
            acc.at[pl.ds(base_rows + _K, _ROWS_PER_TILE - _K)],
        )
        plsc.subcore_barrier()

        def _scale(c, b):
            def body(k16, _):
                kb = k16 * 16
                ew = plsc.bitcast(ebuf[c, 2, pl.ds(kb, 16)], jnp.float32)
                ew = ew * a_s + b_s
                cidx[b][pl.ds(kb, 16)] = ebuf[c, 1, pl.ds(kb, 16)]
                for l in range(16):
                    w = ew[l]
                    for j in range(_DH // 16):
                        sl = pl.ds(j * 16, 16)
                        rows[b][kb + l, sl] = rows[b][kb + l, sl] * w
                return 0

            lax.fori_loop(0, _K // 16, body, 0)

        pltpu.async_copy(g_h.at[ebuf.at[0, 0]], rows0, sg0)
        pltpu.make_async_copy(g_h.at[ebuf.at[0, 0]], rows0, sg0).wait()
        _scale(0, 0)
        pltpu.async_copy(g_h.at[ebuf.at[1, 0]], rows1, sg1)
        pltpu.async_copy(rows0, acc.at[cidx0], ss0, add=True)

        def _pair(c2, _):
            for b2 in range(2):
                c = 1 + 2 * c2 + b2
                b = 1 - b2
                nb = 1 - b
                pltpu.make_async_copy(g_h.at[ebuf.at[c, 0]], rows[b],
                                      gsem[b]).wait()
                _scale(c, b)
                pltpu.make_async_copy(rows[nb], acc.at[cidx[nb]],
                                      ssem[nb]).wait()
                if b2 == 0:
                    pltpu.async_copy(g_h.at[ebuf.at[c + 1, 0]], rows[nb],
                                     gsem[nb])
                else:
                    @pl.when(c2 < _NPAIR - 1)
                    def _start_next():
                        pltpu.async_copy(g_h.at[ebuf.at[c + 1, 0]], rows[nb],
                                         gsem[nb])
                pltpu.async_copy(rows[b], acc.at[cidx[b]], ssem[b], add=True)
            return 0

        lax.fori_loop(0, _NPAIR, _pair, 0)
        pltpu.make_async_copy(rows0, acc.at[cidx0], ss0).wait()
        plsc.subcore_barrier()

        pltpu.sync_copy(
            acc.at[pl.ds(base_rows, _ROWS_PER_TILE)],
            out_h.at[cid, f, pl.ds(base_rows, _ROWS_PER_TILE)],
        )
        if f == 0:
            plsc.subcore_barrier()


@jax.jit
def _conv_agg(g0, g1, edata, ab16):
    k = pl.kernel(
        _conv_agg_body,
        out_type=jax.ShapeDtypeStruct((_NC, 2, _NPAD, _DH), jnp.float32),
        mesh=_sc_mesh(),
        compiler_params=pltpu.CompilerParams(
            use_tc_tiling_on_sc=False, needs_layout_passes=False
        ),
        scratch_types=[
            pltpu.VMEM((_NCHUNK, 3, _K), jnp.int32),
            pltpu.VMEM((_K, _DH), jnp.float32),
            pltpu.VMEM((_K, _DH), jnp.float32),
            pltpu.VMEM((_K,), jnp.int32),
            pltpu.VMEM((_K,), jnp.int32),
            pltpu.VMEM((16,), jnp.float32),
            pltpu.VMEM_SHARED((_NPAD, _DH), jnp.float32),
            pltpu.SemaphoreType.DMA,
            pltpu.SemaphoreType.DMA,
            pltpu.SemaphoreType.DMA,
            pltpu.SemaphoreType.DMA,
        ],
    )
    return k(g0, g1, edata, ab16)


def _edge_w_body(u_h, v_h, eidx_h, edata_h, sc_h, part_h, ebuf, su, sv, cidxb,
                 rawb, rawib, onesb, zb, pbuf, S_acc, C_acc, semu, semv):
    cid = lax.axis_index("c")
    sid = lax.axis_index("s")
    wid = sid * _NC + cid
    base_rows = sid * _ROWS_PER_TILE
    zvec = jnp.zeros((16,), jnp.float32)
    ovec = jnp.ones((16,), jnp.float32)
    lane = lax.iota(jnp.int32, 16)

    pltpu.sync_copy(eidx_h.at[pl.ds(wid * _NCHUNK, _NCHUNK)], ebuf)

    def _fill(i, _):
        zb[pl.ds(i * 16, 16)] = zvec
        onesb[pl.ds(i * 16, 16)] = ovec
        return 0

    lax.fori_loop(0, _K // 16, _fill, 0)

    def _fillz(i, _):
        zb[pl.ds(i * 16, 16)] = zvec
        return 0

    lax.fori_loop(_K // 16, _ROWS_PER_TILE // 16, _fillz, 0)
    pltpu.sync_copy(zb, S_acc.at[pl.ds(base_rows, _ROWS_PER_TILE)])
    pltpu.sync_copy(zb, C_acc.at[pl.ds(base_rows, _ROWS_PER_TILE)])
    plsc.subcore_barrier()

    def _chunk(c, carry):
        vsum, vsq = carry
        pltpu.async_copy(u_h.at[ebuf.at[c, 0]], su, semu)
        pltpu.async_copy(v_h.at[ebuf.at[c, 1]], sv, semv)
        pltpu.make_async_copy(u_h.at[ebuf.at[c, 0]], su, semu).wait()
        pltpu.make_async_copy(v_h.at[ebuf.at[c, 1]], sv, semv).wait()

        def _dot16(k16, carry2):
            vs, vq = carry2
            kb = k16 * 16
            cidxb[pl.ds(kb, 16)] = ebuf[c, 1, pl.ds(kb, 16)]
            rawv = zvec
            for l in range(16):
                m = su[kb + l, pl.ds(0, 16)] * sv[kb + l, pl.ds(0, 16)]
                s = jnp.sum(m)
                rawv = jnp.where(lane == l, s, rawv)
            rawb[pl.ds(kb, 16)] = rawv
            rawib[pl.ds(kb, 16)] = plsc.bitcast(rawv, jnp.int32)
            return (vs + rawv, vq + rawv * rawv)

        carry = lax.fori_loop(0, _K // 16, _dot16, (vsum, vsq))
        off_c = wid * _NCHUNK + c
        pltpu.sync_copy(ebuf.at[c], edata_h.at[off_c, pl.ds(0, 2)])
        pltpu.sync_copy(rawib, edata_h.at[off_c, 2])
        pltpu.sync_copy(rawb, S_acc.at[cidxb], add=True)
        pltpu.sync_copy(onesb, C_acc.at[cidxb], add=True)
        return carry

    vsum, vsq = lax.fori_loop(0, _NCHUNK, _chunk, (zvec, zvec))
    plsc.subcore_barrier()

    pltpu.sync_copy(S_acc.at[pl.ds(base_rows, _ROWS_PER_TILE)],
                    sc_h.at[cid, 0, pl.ds(base_rows, _ROWS_PER_TILE)])
    pltpu.sync_copy(C_acc.at[pl.ds(base_rows, _ROWS_PER_TILE)],
                    sc_h.at[cid, 1, pl.ds(base_rows, _ROWS_PER_TILE)])
    pbuf[0, pl.ds(0, 16)] = vsum
    pbuf[1, pl.ds(0, 16)] = vsq
    pltpu.sync_copy(pbuf, part_h.at[wid])


@jax.jit
def _edge_w(u16, v16, eidx):
    k = pl.kernel(
        _edge_w_body,
        out_type=(
            jax.ShapeDtypeStruct((_NW * _NCHUNK, 3, _K), jnp.int32),
            jax.ShapeDtypeStruct((_NC, 2, _NPAD), jnp.float32),
            jax.ShapeDtypeStruct((_NW, 2, 16), jnp.float32),
        ),
        mesh=_sc_mesh(),
        compiler_params=pltpu.CompilerParams(
            use_tc_tiling_on_sc=False, needs_layout_passes=False
        ),
        scratch_types=[
            pltpu.VMEM((_NCHUNK, 2, _K), jnp.int32),
            pltpu.VMEM((_K, 16), jnp.float32),
            pltpu.VMEM((_K, 16), jnp.float32),
            pltpu.VMEM((_K,), jnp.int32),
            pltpu.VMEM((_K,), jnp.float32),
            pltpu.VMEM((_K,), jnp.int32),
            pltpu.VMEM((_K,), jnp.float32),
            pltpu.VMEM((_ROWS_PER_TILE,), jnp.float32),
            pltpu.VMEM((2, 16), jnp.float32),
            pltpu.VMEM_SHARED((_NPAD,), jnp.float32),
            pltpu.VMEM_SHARED((_NPAD,), jnp.float32),
            pltpu.SemaphoreType.DMA,
            pltpu.SemaphoreType.DMA,
        ],
    )
    return k(u16, v16, eidx)


def _pack_eidx(row, col):
    ed = jnp.stack([row, col])
    ed = ed.reshape(2, _NW, _NCHUNK, _K).transpose(1, 2, 0, 3)
    return ed.reshape(_NW * _NCHUNK, 2, _K)


_BLK = 512
_GRID = _NPAD // _BLK


def _mm(a, b):
    return jnp.dot(a, b, precision=lax.Precision.HIGHEST)


def _full(shape):
    return pl.BlockSpec(shape, lambda i: tuple(0 for _ in shape))


def _rows(shape):
    return pl.BlockSpec(shape, lambda i: (i,) + tuple(0 for _ in shape[1:]))


@jax.jit
def _dense_pre(x_pad, mlp_w1, mlp_b1, mlp_w2, mlp_b2, mlp_w3, mlp_b3,
               parsing0, lin0_w, lin0_b):
    def body(x_ref, w1, b1, w2, b2, w3, b3, p0, l0w, l0b, u_ref, v_ref, h_ref):
        xb = x_ref[...]
        h1 = jnp.maximum(_mm(xb, w1[...]) + b1[...], 0.0)
        h2 = jnp.maximum(_mm(h1, w2[...]) + b2[...], 0.0)
        plv = _mm(h2, w3[...]) + b3[...]
        P = jnp.maximum(_SCALE * p0[...], 0.0)
        vv = _mm(plv, P)
        z = jnp.zeros((_BLK, 8), jnp.float32)
        u_ref[...] = jnp.concatenate([plv, z], axis=1)
        v_ref[...] = jnp.concatenate([vv, z], axis=1)
        h_ref[...] = jnp.maximum(_mm(xb, l0w[...]) + l0b[...], 0.0)

    return pl.pallas_call(
        body,
        grid=(_GRID,),
        in_specs=[
            _rows((_BLK, _D)),
            _full((_D, 512)), _full((1, 512)),
            _full((512, 64)), _full((1, 64)),
            _full((64, 8)), _full((1, 8)),
            _full((8, 8)),
            _full((_D, _D)), _full((1, _D)),
        ],
        out_specs=[_rows((_BLK, 16)), _rows((_BLK, 16)), _rows((_BLK, _D))],
        out_shape=[
            jax.ShapeDtypeStruct((_NPAD, 16), jnp.float32),
            jax.ShapeDtypeStruct((_NPAD, 16), jnp.float32),
            jax.ShapeDtypeStruct((_NPAD, _D), jnp.float32),
        ],
    )(x_pad, mlp_w1, mlp_b1.reshape(1, -1), mlp_w2, mlp_b2.reshape(1, -1),
      mlp_w3, mlp_b3.reshape(1, -1), parsing0, lin0_w, lin0_b.reshape(1, -1))


def _prep0(sc_parts, mv_parts, h0):
    def body(mv_ref, sc_ref, h_ref, ab_ref, d_ref, g0_ref, g1_ref):
        mv = mv_ref[...]
        sum_r = jnp.sum(mv[:, 0, :])
        sum_q = jnp.sum(mv[:, 1, :])
        mean = sum_r / _E
        var = (sum_q - _E * mean * mean) / (_E - 1)
        a = jnp.sqrt(_EPSV / var)
        b = 1.0 - a * mean
        lane = lax.broadcasted_iota(jnp.int32, (1, 16), 1)
        ab_ref[...] = jnp.where(lane == 0, a, jnp.where(lane == 1, b, 0.0))
        sc = sc_ref[...]
        S = sc[0, 0] + sc[1, 0]
        C = sc[0, 1] + sc[1, 1]
        deg = 1.0 + a * S + b * C
        dinv = jnp.where(deg > 0, lax.rsqrt(jnp.where(deg > 0, deg, 1.0)), 0.0)
        d_ref[...] = dinv
        g = h_ref[...] * dinv[:, None]
        g0_ref[...] = g[:, : _DH]
        g1_ref[...] = g[:, _DH:]

    return pl.pallas_call(
        body,
        grid=(_GRID,),
        in_specs=[
            _full((_NW, 2, 16)),
            pl.BlockSpec((_NC, 2, _BLK), lambda i: (0, 0, i)),
            _rows((_BLK, _D)),
        ],
        out_specs=[
            pl.BlockSpec((1, 16), lambda i: (0, 0)),
            _rows((_BLK,)),
            _rows((_BLK, _DH)),
            _rows((_BLK, _DH)),
        ],
        out_shape=[
            jax.ShapeDtypeStruct((1, 16), jnp.float32),
            jax.ShapeDtypeStruct((_NPAD,), jnp.float32),
            jax.ShapeDtypeStruct((_NPAD, _DH), jnp.float32),
            jax.ShapeDtypeStruct((_NPAD, _DH), jnp.float32),
        ],
    )(mv_parts, sc_parts, h0)


def _conv_mix_g(parts, dinv2, h, w1, beta):
    def body(p_ref, d_ref, h_ref, w_ref, o_ref, g0_ref, g1_ref):
        p = p_ref[...]
        agg = jnp.concatenate([p[0, 0] + p[1, 0], p[0, 1] + p[1, 1]], axis=1)
        dv = d_ref[...][:, None]
        agg = agg * dv + (dv * dv) * h_ref[...]
        out = (1.0 - beta) * agg + beta * _mm(agg, w_ref[...])
        hn = jnp.maximum(out, 0.0)
        o_ref[...] = hn
        g = hn * dv
        g0_ref[...] = g[:, : _DH]
        g1_ref[...] = g[:, _DH:]

    return pl.pallas_call(
        body,
        grid=(_GRID,),
        in_specs=[
            pl.BlockSpec((_NC, 2, _BLK, _DH), lambda i: (0, 0, i, 0)),
            _rows((_BLK,)),
            _rows((_BLK, _D)),
            _full((_D, _D)),
        ],
        out_specs=[_rows((_BLK, _D)), _rows((_BLK, _DH)), _rows((_BLK, _DH))],
        out_shape=[
            jax.ShapeDtypeStruct((_NPAD, _D), jnp.float32),
            jax.ShapeDtypeStruct((_NPAD, _DH), jnp.float32),
            jax.ShapeDtypeStruct((_NPAD, _DH), jnp.float32),
        ],
    )(parts, dinv2, h, w1)


def _conv_mix_final(parts, dinv2, h, w1, beta, lin1_w, lin1_b):
    def body(p_ref, d_ref, h_ref, w_ref, lw_ref, lb_ref, o_ref):
        p = p_ref[...]
        agg = jnp.concatenate([p[0, 0] + p[1, 0], p[0, 1] + p[1, 1]], axis=1)
        dv = d_ref[...][:, None]
        agg = agg * dv + (dv * dv) * h_ref[...]
        out = (1.0 - beta) * agg + beta * _mm(agg, w_ref[...])
        hn = jnp.maximum(out, 0.0)
        o_ref[...] = _mm(hn, lw_ref[...]) + lb_ref[...]

    return pl.pallas_call(
        body,
        grid=(_GRID,),
        in_specs=[
            pl.BlockSpec((_NC, 2, _BLK, _DH), lambda i: (0, 0, i, 0)),
            _rows((_BLK,)),
            _rows((_BLK, _D)),
            _full((_D, _D)),
            _full((_D, 8)),
            _full((1, 8)),
        ],
        out_specs=_rows((_BLK, 8)),
        out_shape=jax.ShapeDtypeStruct((_NPAD, 8), jnp.float32),
    )(parts, dinv2, h, w1, lin1_w, lin1_b.reshape(1, -1))


def kernel(x, edge_index, lin0_w, lin0_b, lin1_w, lin1_b, mlp_w1, mlp_b1,
           mlp_w2, mlp_b2, mlp_w3, mlp_b3, parsing0, parsing1, conv0_w1,
           conv0_w2, conv1_w1, conv1_w2):
    row = edge_index[0]
    col = edge_index[1]
    x_pad = jnp.zeros((_NPAD, _D), jnp.float32).at[: _N].set(x)

    u16, v16, h0 = _dense_pre(x_pad, mlp_w1, mlp_b1, mlp_w2, mlp_b2,
                              mlp_w3, mlp_b3, parsing0, lin0_w, lin0_b)

    eidx = _pack_eidx(row, col)
    edata, sc_parts, mv_parts = _edge_w(u16, v16, eidx)

    ab2d, dinv2, g0, g1 = _prep0(sc_parts, mv_parts, h0)
    ab16 = ab2d[0]

    beta0 = math.log(_THETA + 1.0)
    beta1 = math.log(_THETA / 2 + 1.0)
    parts = _conv_agg(g0, g1, edata, ab16)
    h1, g0, g1 = _conv_mix_g(parts, dinv2, h0, conv0_w1, beta0)
    parts = _conv_agg(g0, g1, edata, ab16)
    return _conv_mix_final(parts, dinv2, h1, conv1_w1, beta1, lin1_w,
                           lin1_b)[: _N]
